# Initial kernel scaffold; baseline (speedup 1.0000x reference)
import functools

import numpy as np
import jax
import jax.numpy as jnp
from jax import lax
from jax.experimental import pallas as pl
from jax.experimental.pallas import tpu as pltpu
from jax.experimental.pallas import tpu_sc as plsc

N_POINTS = 1048576
N_LEVELS = 12
LOG2_T = 19
T = 1 << LOG2_T
BASE_RES = 16
SCALE = 1.3819
WIDTH = 64
N_OUT = 16
PRIME1 = 2654435761
PRIME2 = 805459861

_RES = [int(np.floor(BASE_RES * (SCALE ** l))) for l in range(N_LEVELS)]
_DENSE = [(r + 1) ** 3 <= T for r in _RES]

NC = 2
NS = 16
NW = NC * NS
PPW = N_POINTS // NW
C = 1024
G = C // 16
NCHUNK = PPW // C


def _sc_encode(x0, x1, x2, *tabs):
    mesh = plsc.VectorSubcoreMesh(core_axis_name="c", subcore_axis_name="s")

    @functools.partial(
        pl.kernel,
        out_type=jax.ShapeDtypeStruct((N_POINTS, 2 * N_LEVELS), jnp.float32),
        mesh=mesh,
        scratch_types=[
            pltpu.VMEM((3, C), jnp.float32),
            pltpu.VMEM((2, 8 * C), jnp.int32),
            pltpu.VMEM((2, 8 * C, 2), jnp.float32),
            pltpu.VMEM((C, 2 * N_LEVELS), jnp.float32),
            pltpu.SemaphoreType.DMA,
            pltpu.SemaphoreType.DMA,
        ],
    )
    def enc(*refs):
        x_refs = refs[:3]
        tab_refs = refs[3:3 + N_LEVELS]
        pe_h = refs[3 + N_LEVELS]
        xb, idxb, landb, peb, sem0, sem1 = refs[4 + N_LEVELS:]
        sems = (sem0, sem1)

        wid = lax.axis_index("s") * NC + lax.axis_index("c")
        iota = lax.iota(jnp.int32, 16)
        zero16 = jnp.zeros((16,), jnp.int32)
        one16 = jnp.full((16,), 1, jnp.int32)
        inv256 = jnp.float32(1.0 / 256.0)

        def load_xyz(g, resf):
            base = g * 16
            xs = xb[0, pl.ds(base, 16)] * inv256 * resf
            ys = xb[1, pl.ds(base, 16)] * inv256 * resf
            zs = xb[2, pl.ds(base, 16)] * inv256 * resf
            return xs, ys, zs

        def phase_a(l, s):
            resf = jnp.float32(_RES[l])
            rm1 = jnp.float32(_RES[l] - 1)

            def body(g, carry):
                xs, ys, zs = load_xyz(g, resf)
                cx = jnp.clip(jnp.floor(xs), 0.0, rm1).astype(jnp.int32)
                cy = jnp.clip(jnp.floor(ys), 0.0, rm1).astype(jnp.int32)
                cz = jnp.clip(jnp.floor(zs), 0.0, rm1).astype(jnp.int32)
                if _DENSE[l]:
                    s1 = _RES[l] + 1
                    ax = (cx, cx + 1)
                    by0 = cy * s1
                    by = (by0, by0 + s1)
                    cz0 = cz * (s1 * s1)
                    czc = (cz0, cz0 + s1 * s1)
                    ids = [ax[i] + by[j] + czc[k]
                           for i in (0, 1) for j in (0, 1) for k in (0, 1)]
                else:
                    ux = cx.astype(jnp.uint32)
                    uy = cy.astype(jnp.uint32)
                    uz = cz.astype(jnp.uint32)
                    hx = (ux, ux + jnp.uint32(1))
                    hy0 = uy * jnp.uint32(PRIME1)
                    hy = (hy0, hy0 + jnp.uint32(PRIME1))
                    hz0 = uz * jnp.uint32(PRIME2)
                    hz = (hz0, hz0 + jnp.uint32(PRIME2))
                    mask = jnp.uint32(T - 1)
                    ids = [((hx[i] ^ hy[j] ^ hz[k]) & mask).astype(jnp.int32)
                           for i in (0, 1) for j in (0, 1) for k in (0, 1)]
                for c8 in range(8):
                    idxb[s, pl.ds((g * 8 + c8) * 16, 16)] = ids[c8]
                return carry

            lax.fori_loop(0, G, body, 0)

        def issue(l, s):
            return pltpu.async_copy(tab_refs[l].at[idxb.at[s]], landb.at[s], sems[s])

        def phase_b(l, s):
            resf = jnp.float32(_RES[l])
            col0 = jnp.full((16,), 2 * l, jnp.int32)
            col1 = jnp.full((16,), 2 * l + 1, jnp.int32)

            def body(g, carry):
                xs, ys, zs = load_xyz(g, resf)
                wx1 = xs - jnp.floor(xs)
                wy1 = ys - jnp.floor(ys)
                wz1 = zs - jnp.floor(zs)
                wx = (jnp.float32(1.0) - wx1, wx1)
                wy = (jnp.float32(1.0) - wy1, wy1)
                wz = (jnp.float32(1.0) - wz1, wz1)
                m = [wx[i] * wy[j] for i in (0, 1) for j in (0, 1)]
                acc0 = jnp.zeros((16,), jnp.float32)
                acc1 = jnp.zeros((16,), jnp.float32)
                rbase = g * 128
                for c8 in range(8):
                    i = (c8 >> 2) & 1
                    j = (c8 >> 1) & 1
                    k = c8 & 1
                    wc = m[i * 2 + j] * wz[k]
                    rvec = iota + (rbase + c8 * 16)
                    f0 = plsc.load_gather(landb.at[s], [rvec, zero16])
                    f1 = plsc.load_gather(landb.at[s], [rvec, one16])
                    acc0 = acc0 + f0 * wc
                    acc1 = acc1 + f1 * wc
                pvec = iota + g * 16
                plsc.store_scatter(peb, [pvec, col0], acc0)
                plsc.store_scatter(peb, [pvec, col1], acc1)
                return carry

            lax.fori_loop(0, G, body, 0)

        def chunk_body(ci, carry):
            rowbase = wid * PPW + ci * C
            pltpu.sync_copy(x_refs[0].at[pl.ds(rowbase, C)], xb.at[0])
            pltpu.sync_copy(x_refs[1].at[pl.ds(rowbase, C)], xb.at[1])
            pltpu.sync_copy(x_refs[2].at[pl.ds(rowbase, C)], xb.at[2])
            phase_a(0, 0)
            cp = issue(0, 0)
            for l in range(N_LEVELS):
                s = l % 2
                nxt = None
                if l + 1 < N_LEVELS:
                    phase_a(l + 1, 1 - s)
                    nxt = issue(l + 1, 1 - s)
                cp.wait()
                phase_b(l, s)
                cp = nxt
            pltpu.sync_copy(peb, pe_h.at[pl.ds(rowbase, C)])
            return carry

        lax.fori_loop(0, NCHUNK, chunk_body, 0)

    return enc(x0, x1, x2, *tabs)


def _tc_mlp(pe, W0, W1, W2):
    B = 2048

    def body(pe_ref, w0_ref, w1_ref, w2_ref, z_ref, d_ref):
        h = jnp.maximum(
            jnp.dot(pe_ref[...], w0_ref[...], preferred_element_type=jnp.float32), 0.0)
        h = jnp.maximum(
            jnp.dot(h, w1_ref[...], preferred_element_type=jnp.float32), 0.0)
        z = jnp.dot(h, w2_ref[...], preferred_element_type=jnp.float32)
        z_ref[...] = z
        z0 = z[:, 0]
        d_ref[...] = jnp.maximum(z0, 0.0) + jnp.log1p(jnp.exp(-jnp.abs(z0)))

    return pl.pallas_call(
        body,
        grid=(N_POINTS // B,),
        in_specs=[
            pl.BlockSpec((B, 2 * N_LEVELS), lambda i: (i, 0)),
            pl.BlockSpec((2 * N_LEVELS, WIDTH), lambda i: (0, 0)),
            pl.BlockSpec((WIDTH, WIDTH), lambda i: (0, 0)),
            pl.BlockSpec((WIDTH, N_OUT), lambda i: (0, 0)),
        ],
        out_specs=[
            pl.BlockSpec((B, N_OUT), lambda i: (i, 0)),
            pl.BlockSpec((B,), lambda i: (i,)),
        ],
        out_shape=[
            jax.ShapeDtypeStruct((N_POINTS, N_OUT), jnp.float32),
            jax.ShapeDtypeStruct((N_POINTS,), jnp.float32),
        ],
    )(pe, W0, W1, W2)


def kernel(x, tables, W0, W1, W2):
    x0 = x[:, 0]
    x1 = x[:, 1]
    x2 = x[:, 2]
    tabs = [tables[l] for l in range(N_LEVELS)]
    pe = _sc_encode(x0, x1, x2, *tabs)
    z, density = _tc_mlp(pe, W0, W1, W2)
    return (density, pe, z)


# speedup vs baseline: 62.6420x; 62.6420x over previous
import functools

import numpy as np
import jax
import jax.numpy as jnp
from jax import lax
from jax.experimental import pallas as pl
from jax.experimental.pallas import tpu as pltpu
from jax.experimental.pallas import tpu_sc as plsc

N_POINTS = 1048576
N_LEVELS = 12
LOG2_T = 19
T = 1 << LOG2_T
BASE_RES = 16
SCALE = 1.3819
WIDTH = 64
N_OUT = 16
PRIME1 = 2654435761
PRIME2 = 805459861

_RES = [int(np.floor(BASE_RES * (SCALE ** l))) for l in range(N_LEVELS)]
_DENSE = [(r + 1) ** 3 <= T for r in _RES]

NC = 2
NS = 16
NW = NC * NS
PPW = N_POINTS // NW
C = 1024
G = C // 16
NCHUNK = PPW // C


def _sc_encode(x0, x1, x2, *tabs):
    mesh = plsc.VectorSubcoreMesh(core_axis_name="c", subcore_axis_name="s")

    @functools.partial(
        pl.kernel,
        out_type=jax.ShapeDtypeStruct((2 * N_LEVELS, N_POINTS), jnp.float32),
        mesh=mesh,
        scratch_types=[
            pltpu.VMEM((3 * C,), jnp.float32),
            pltpu.VMEM((16 * C,), jnp.int32),
            pltpu.VMEM((16 * C,), jnp.int32),
            pltpu.VMEM((16 * C,), jnp.float32),
            pltpu.VMEM((16 * C,), jnp.float32),
            pltpu.VMEM((2 * N_LEVELS, C), jnp.float32),
            pltpu.SemaphoreType.DMA,
            pltpu.SemaphoreType.DMA,
        ],
    )
    def enc(*refs):
        x_refs = refs[:3]
        tab_refs = refs[3:3 + N_LEVELS]
        pe_h = refs[3 + N_LEVELS]
        xb, idx0, idx1, land0, land1, peb, sem0, sem1 = refs[4 + N_LEVELS:]
        idxs = (idx0, idx1)
        lands = (land0, land1)
        sems = (sem0, sem1)

        wid = lax.axis_index("s") * NC + lax.axis_index("c")
        inv256 = jnp.float32(1.0 / 256.0)

        def load_xyz(g, resf):
            base = g * 16
            xs = xb[pl.ds(base, 16)] * inv256 * resf
            ys = xb[pl.ds(C + base, 16)] * inv256 * resf
            zs = xb[pl.ds(2 * C + base, 16)] * inv256 * resf
            return xs, ys, zs

        def phase_a(l, s):
            resf = jnp.float32(_RES[l])
            rm1 = jnp.float32(_RES[l] - 1)

            def body(g, carry):
                xs, ys, zs = load_xyz(g, resf)
                cx = xs.astype(jnp.int32)
                cy = ys.astype(jnp.int32)
                cz = zs.astype(jnp.int32)
                if _DENSE[l]:
                    s1 = _RES[l] + 1
                    ax = (cx, cx + 1)
                    by0 = cy * s1
                    by = (by0, by0 + s1)
                    cz0 = cz * (s1 * s1)
                    czc = (cz0, cz0 + s1 * s1)
                    ids = [ax[i] + by[j] + czc[k]
                           for i in (0, 1) for j in (0, 1) for k in (0, 1)]
                else:
                    ux = cx.astype(jnp.uint32)
                    uy = cy.astype(jnp.uint32)
                    uz = cz.astype(jnp.uint32)
                    hx = (ux, ux + jnp.uint32(1))
                    hy0 = uy * jnp.uint32(PRIME1)
                    hy = (hy0, hy0 + jnp.uint32(PRIME1))
                    hz0 = uz * jnp.uint32(PRIME2)
                    hz = (hz0, hz0 + jnp.uint32(PRIME2))
                    mask = jnp.uint32(T - 1)
                    ids = [((hx[i] ^ hy[j] ^ hz[k]) & mask).astype(jnp.int32)
                           for i in (0, 1) for j in (0, 1) for k in (0, 1)]
                for c8 in range(8):
                    w0 = ids[c8] * 2
                    idxs[s][pl.ds((g * 8 + c8) * 32, 16)] = w0
                    idxs[s][pl.ds((g * 8 + c8) * 32 + 16, 16)] = w0 + 1
                return carry

            lax.fori_loop(0, G, body, 0)

        def issue(l, s):
            return pltpu.async_copy(tab_refs[l].at[idxs[s]], lands[s], sems[s])

        def phase_b(l, s):
            resf = jnp.float32(_RES[l])

            def body(g, carry):
                xs, ys, zs = load_xyz(g, resf)
                wx1 = xs - xs.astype(jnp.int32).astype(jnp.float32)
                wy1 = ys - ys.astype(jnp.int32).astype(jnp.float32)
                wz1 = zs - zs.astype(jnp.int32).astype(jnp.float32)
                wx = (jnp.float32(1.0) - wx1, wx1)
                wy = (jnp.float32(1.0) - wy1, wy1)
                wz = (jnp.float32(1.0) - wz1, wz1)
                m = [wx[i] * wy[j] for i in (0, 1) for j in (0, 1)]
                acc0 = jnp.zeros((16,), jnp.float32)
                acc1 = jnp.zeros((16,), jnp.float32)
                rbase = g * 256
                for c8 in range(8):
                    i = (c8 >> 2) & 1
                    j = (c8 >> 1) & 1
                    k = c8 & 1
                    wc = m[i * 2 + j] * wz[k]
                    f0 = lands[s][pl.ds(rbase + c8 * 32, 16)]
                    f1 = lands[s][pl.ds(rbase + c8 * 32 + 16, 16)]
                    acc0 = acc0 + f0 * wc
                    acc1 = acc1 + f1 * wc
                peb[2 * l, pl.ds(g * 16, 16)] = acc0
                peb[2 * l + 1, pl.ds(g * 16, 16)] = acc1
                return carry

            lax.fori_loop(0, G, body, 0)

        def chunk_body(ci, carry):
            rowbase = wid * PPW + ci * C
            pltpu.sync_copy(x_refs[0].at[pl.ds(rowbase, C)], xb.at[pl.ds(0, C)])
            pltpu.sync_copy(x_refs[1].at[pl.ds(rowbase, C)], xb.at[pl.ds(C, C)])
            pltpu.sync_copy(x_refs[2].at[pl.ds(rowbase, C)], xb.at[pl.ds(2 * C, C)])
            phase_a(0, 0)
            cp = issue(0, 0)
            for l in range(N_LEVELS):
                s = l % 2
                nxt = None
                if l + 1 < N_LEVELS:
                    phase_a(l + 1, 1 - s)
                    nxt = issue(l + 1, 1 - s)
                cp.wait()
                phase_b(l, s)
                cp = nxt
            pltpu.sync_copy(peb, pe_h.at[:, pl.ds(rowbase, C)])
            return carry

        lax.fori_loop(0, NCHUNK, chunk_body, 0)

    return enc(x0, x1, x2, *tabs)


def _tc_mlp(peT, W0, W1, W2):
    B = 2048

    def body(pet_ref, w0_ref, w1_ref, w2_ref, pe_ref, z_ref, d_ref):
        p = pet_ref[...].T
        pe_ref[...] = p
        h = jnp.maximum(
            jnp.dot(p, w0_ref[...], preferred_element_type=jnp.float32), 0.0)
        h = jnp.maximum(
            jnp.dot(h, w1_ref[...], preferred_element_type=jnp.float32), 0.0)
        z = jnp.dot(h, w2_ref[...], preferred_element_type=jnp.float32)
        z_ref[...] = z
        z0 = z[:, 0]
        d_ref[...] = jnp.maximum(z0, 0.0) + jnp.log1p(jnp.exp(-jnp.abs(z0)))

    return pl.pallas_call(
        body,
        grid=(N_POINTS // B,),
        in_specs=[
            pl.BlockSpec((2 * N_LEVELS, B), lambda i: (0, i)),
            pl.BlockSpec((2 * N_LEVELS, WIDTH), lambda i: (0, 0)),
            pl.BlockSpec((WIDTH, WIDTH), lambda i: (0, 0)),
            pl.BlockSpec((WIDTH, N_OUT), lambda i: (0, 0)),
        ],
        out_specs=[
            pl.BlockSpec((B, 2 * N_LEVELS), lambda i: (i, 0)),
            pl.BlockSpec((B, N_OUT), lambda i: (i, 0)),
            pl.BlockSpec((B,), lambda i: (i,)),
        ],
        out_shape=[
            jax.ShapeDtypeStruct((N_POINTS, 2 * N_LEVELS), jnp.float32),
            jax.ShapeDtypeStruct((N_POINTS, N_OUT), jnp.float32),
            jax.ShapeDtypeStruct((N_POINTS,), jnp.float32),
        ],
    )(peT, W0, W1, W2)


def kernel(x, tables, W0, W1, W2):
    x0 = x[:, 0]
    x1 = x[:, 1]
    x2 = x[:, 2]
    tabs = [tables[l].reshape(-1) for l in range(N_LEVELS)]
    peT = _sc_encode(x0, x1, x2, *tabs)
    pe, z, density = _tc_mlp(peT, W0, W1, W2)
    return (density, pe, z)


# speedup vs baseline: 82.3527x; 1.3147x over previous
import functools

import numpy as np
import jax
import jax.numpy as jnp
from jax import lax
from jax.experimental import pallas as pl
from jax.experimental.pallas import tpu as pltpu
from jax.experimental.pallas import tpu_sc as plsc

N_POINTS = 1048576
N_LEVELS = 12
LOG2_T = 19
T = 1 << LOG2_T
BASE_RES = 16
SCALE = 1.3819
WIDTH = 64
N_OUT = 16
PRIME1 = 2654435761
PRIME2 = 805459861

_RES = [int(np.floor(BASE_RES * (SCALE ** l))) for l in range(N_LEVELS)]
_DENSE = [(r + 1) ** 3 <= T for r in _RES]
_N_STAGED = min(sum(_DENSE), 4)
_STAGE_SZ = [-(-2 * (_RES[l] + 1) ** 3 // 8) * 8 for l in range(_N_STAGED)]

NC = 2
NS = 16
NW = NC * NS
PPW = N_POINTS // NW
C = 1024
G = C // 16
NCHUNK = PPW // C


def _sc_encode(x0, x1, x2, *tabs):
    mesh = plsc.VectorSubcoreMesh(core_axis_name="c", subcore_axis_name="s")

    @functools.partial(
        pl.kernel,
        out_type=jax.ShapeDtypeStruct((2 * N_LEVELS, N_POINTS), jnp.float32),
        mesh=mesh,
        scratch_types=[
            pltpu.VMEM((3 * C,), jnp.float32),
            pltpu.VMEM((16 * C,), jnp.int32),
            pltpu.VMEM((16 * C,), jnp.int32),
            pltpu.VMEM((16 * C,), jnp.float32),
            pltpu.VMEM((16 * C,), jnp.float32),
            pltpu.VMEM((2 * N_LEVELS, C), jnp.float32),
            pltpu.SemaphoreType.DMA,
            pltpu.SemaphoreType.DMA,
        ] + [pltpu.VMEM_SHARED((sz,), jnp.float32) for sz in _STAGE_SZ],
    )
    def enc(*refs):
        x_refs = refs[:3]
        tab_refs = refs[3:3 + N_LEVELS]
        pe_h = refs[3 + N_LEVELS]
        xb, idx0, idx1, land0, land1, peb, sem0, sem1 = refs[4 + N_LEVELS:4 + N_LEVELS + 8]
        sp_refs = refs[4 + N_LEVELS + 8:]
        idxs = (idx0, idx1)
        lands = (land0, land1)
        sems = (sem0, sem1)

        wid = lax.axis_index("s") * NC + lax.axis_index("c")
        inv256 = jnp.float32(1.0 / 256.0)

        def load_xyz(g, resf):
            base = g * 16
            xs = xb[pl.ds(base, 16)] * inv256 * resf
            ys = xb[pl.ds(C + base, 16)] * inv256 * resf
            zs = xb[pl.ds(2 * C + base, 16)] * inv256 * resf
            return xs, ys, zs

        def phase_a(l, s):
            resf = jnp.float32(_RES[l])
            rm1 = jnp.float32(_RES[l] - 1)

            def body(g, carry):
                xs, ys, zs = load_xyz(g, resf)
                cx = xs.astype(jnp.int32)
                cy = ys.astype(jnp.int32)
                cz = zs.astype(jnp.int32)
                if _DENSE[l]:
                    s1 = _RES[l] + 1
                    ax = (cx, cx + 1)
                    by0 = cy * s1
                    by = (by0, by0 + s1)
                    cz0 = cz * (s1 * s1)
                    czc = (cz0, cz0 + s1 * s1)
                    ids = [ax[i] + by[j] + czc[k]
                           for i in (0, 1) for j in (0, 1) for k in (0, 1)]
                else:
                    ux = cx.astype(jnp.uint32)
                    uy = cy.astype(jnp.uint32)
                    uz = cz.astype(jnp.uint32)
                    hx = (ux, ux + jnp.uint32(1))
                    hy0 = uy * jnp.uint32(PRIME1)
                    hy = (hy0, hy0 + jnp.uint32(PRIME1))
                    hz0 = uz * jnp.uint32(PRIME2)
                    hz = (hz0, hz0 + jnp.uint32(PRIME2))
                    mask = jnp.uint32(T - 1)
                    ids = [((hx[i] ^ hy[j] ^ hz[k]) & mask).astype(jnp.int32)
                           for i in (0, 1) for j in (0, 1) for k in (0, 1)]
                for c8 in range(8):
                    w0 = ids[c8] * 2
                    idxs[s][pl.ds((g * 8 + c8) * 32, 16)] = w0
                    idxs[s][pl.ds((g * 8 + c8) * 32 + 16, 16)] = w0 + 1
                return carry

            lax.fori_loop(0, G, body, 0)

        def issue(l, s):
            src_ref = sp_refs[l] if l < _N_STAGED else tab_refs[l]
            return pltpu.async_copy(src_ref.at[idxs[s]], lands[s], sems[s])

        def phase_b(l, s):
            resf = jnp.float32(_RES[l])

            def body(g, carry):
                xs, ys, zs = load_xyz(g, resf)
                wx1 = xs - xs.astype(jnp.int32).astype(jnp.float32)
                wy1 = ys - ys.astype(jnp.int32).astype(jnp.float32)
                wz1 = zs - zs.astype(jnp.int32).astype(jnp.float32)
                wx = (jnp.float32(1.0) - wx1, wx1)
                wy = (jnp.float32(1.0) - wy1, wy1)
                wz = (jnp.float32(1.0) - wz1, wz1)
                m = [wx[i] * wy[j] for i in (0, 1) for j in (0, 1)]
                acc0 = jnp.zeros((16,), jnp.float32)
                acc1 = jnp.zeros((16,), jnp.float32)
                rbase = g * 256
                for c8 in range(8):
                    i = (c8 >> 2) & 1
                    j = (c8 >> 1) & 1
                    k = c8 & 1
                    wc = m[i * 2 + j] * wz[k]
                    f0 = lands[s][pl.ds(rbase + c8 * 32, 16)]
                    f1 = lands[s][pl.ds(rbase + c8 * 32 + 16, 16)]
                    acc0 = acc0 + f0 * wc
                    acc1 = acc1 + f1 * wc
                peb[2 * l, pl.ds(g * 16, 16)] = acc0
                peb[2 * l + 1, pl.ds(g * 16, 16)] = acc1
                return carry

            lax.fori_loop(0, G, body, 0)

        def chunk_body(ci, carry):
            rowbase = wid * PPW + ci * C
            pltpu.sync_copy(x_refs[0].at[pl.ds(rowbase, C)], xb.at[pl.ds(0, C)])
            pltpu.sync_copy(x_refs[1].at[pl.ds(rowbase, C)], xb.at[pl.ds(C, C)])
            pltpu.sync_copy(x_refs[2].at[pl.ds(rowbase, C)], xb.at[pl.ds(2 * C, C)])
            phase_a(0, 0)
            cp = issue(0, 0)
            for l in range(N_LEVELS):
                s = l % 2
                nxt = None
                if l + 1 < N_LEVELS:
                    phase_a(l + 1, 1 - s)
                    nxt = issue(l + 1, 1 - s)
                cp.wait()
                phase_b(l, s)
                cp = nxt
            pltpu.sync_copy(peb, pe_h.at[:, pl.ds(rowbase, C)])
            return carry

        s_idx = lax.axis_index("s")
        piece = 0
        for l in range(_N_STAGED):
            off = 0
            while off < _STAGE_SZ[l]:
                n = min(16 * C, _STAGE_SZ[l] - off)

                def _stage(l=l, off=off, n=n):
                    pltpu.sync_copy(tab_refs[l].at[pl.ds(off, n)],
                                    land0.at[pl.ds(0, n)])
                    pltpu.sync_copy(land0.at[pl.ds(0, n)],
                                    sp_refs[l].at[pl.ds(off, n)])

                pl.when(s_idx == (piece % NS))(_stage)
                piece += 1
                off += n
        plsc.subcore_barrier()
        lax.fori_loop(0, NCHUNK, chunk_body, 0)

    return enc(x0, x1, x2, *tabs)


def _tc_mlp(peT, W0, W1, W2):
    B = 2048

    def body(pet_ref, w0_ref, w1_ref, w2_ref, pe_ref, z_ref, d_ref):
        p = pet_ref[...].T
        pe_ref[...] = p
        h = jnp.maximum(
            jnp.dot(p, w0_ref[...], preferred_element_type=jnp.float32), 0.0)
        h = jnp.maximum(
            jnp.dot(h, w1_ref[...], preferred_element_type=jnp.float32), 0.0)
        z = jnp.dot(h, w2_ref[...], preferred_element_type=jnp.float32)
        z_ref[...] = z
        z0 = z[:, 0]
        d_ref[...] = jnp.maximum(z0, 0.0) + jnp.log1p(jnp.exp(-jnp.abs(z0)))

    return pl.pallas_call(
        body,
        grid=(N_POINTS // B,),
        in_specs=[
            pl.BlockSpec((2 * N_LEVELS, B), lambda i: (0, i)),
            pl.BlockSpec((2 * N_LEVELS, WIDTH), lambda i: (0, 0)),
            pl.BlockSpec((WIDTH, WIDTH), lambda i: (0, 0)),
            pl.BlockSpec((WIDTH, N_OUT), lambda i: (0, 0)),
        ],
        out_specs=[
            pl.BlockSpec((B, 2 * N_LEVELS), lambda i: (i, 0)),
            pl.BlockSpec((B, N_OUT), lambda i: (i, 0)),
            pl.BlockSpec((B,), lambda i: (i,)),
        ],
        out_shape=[
            jax.ShapeDtypeStruct((N_POINTS, 2 * N_LEVELS), jnp.float32),
            jax.ShapeDtypeStruct((N_POINTS, N_OUT), jnp.float32),
            jax.ShapeDtypeStruct((N_POINTS,), jnp.float32),
        ],
    )(peT, W0, W1, W2)


def kernel(x, tables, W0, W1, W2):
    x0 = x[:, 0]
    x1 = x[:, 1]
    x2 = x[:, 2]
    tabs = [tables[l].reshape(-1) for l in range(N_LEVELS)]
    peT = _sc_encode(x0, x1, x2, *tabs)
    pe, z, density = _tc_mlp(peT, W0, W1, W2)
    return (density, pe, z)


# speedup vs baseline: 211.1647x; 2.5641x over previous
import functools

import numpy as np
import jax
import jax.numpy as jnp
from jax import lax
from jax.experimental import pallas as pl
from jax.experimental.pallas import tpu as pltpu
from jax.experimental.pallas import tpu_sc as plsc

N_POINTS = 1048576
N_LEVELS = 12
LOG2_T = 19
T = 1 << LOG2_T
BASE_RES = 16
SCALE = 1.3819
WIDTH = 64
N_OUT = 16
PRIME1 = 2654435761
PRIME2 = 805459861

_RES = [int(np.floor(BASE_RES * (SCALE ** l))) for l in range(N_LEVELS)]
_DENSE = [(r + 1) ** 3 <= T for r in _RES]
_N_STAGED = 6
_ROWS = [min((_RES[l] + 1) ** 3, T) if _DENSE[l] else T for l in range(N_LEVELS)]
_STAGE_SZ = [-(-_ROWS[l] // 8) * 8 for l in range(_N_STAGED)]

NC = 2
NS = 16
NW = NC * NS
PPW = N_POINTS // NW
C = 1024
G = C // 16
NCHUNK = PPW // C


def _sc_encode(x0, x1, x2, *tabs):
    mesh = plsc.VectorSubcoreMesh(core_axis_name="c", subcore_axis_name="s")

    @functools.partial(
        pl.kernel,
        out_type=jax.ShapeDtypeStruct((2 * N_LEVELS, N_POINTS), jnp.float32),
        mesh=mesh,
        scratch_types=[
            pltpu.VMEM((3 * C,), jnp.float32),
            pltpu.VMEM((8 * C,), jnp.int32),
            pltpu.VMEM((8 * C,), jnp.int32),
            pltpu.VMEM((8 * C,), jnp.int32),
            pltpu.VMEM((8 * C,), jnp.int32),
            pltpu.VMEM((2 * N_LEVELS, C), jnp.float32),
            pltpu.SemaphoreType.DMA,
            pltpu.SemaphoreType.DMA,
        ] + [pltpu.VMEM_SHARED((sz,), jnp.int32) for sz in _STAGE_SZ],
    )
    def enc(*refs):
        x_refs = refs[:3]
        tab_refs = refs[3:3 + N_LEVELS]
        pe_h = refs[3 + N_LEVELS]
        xb, idx0, idx1, land0, land1, peb, sem0, sem1 = refs[4 + N_LEVELS:4 + N_LEVELS + 8]
        sp_refs = refs[4 + N_LEVELS + 8:]
        idxs = (idx0, idx1)
        lands = (land0, land1)
        sems = (sem0, sem1)

        wid = lax.axis_index("s") * NC + lax.axis_index("c")
        inv256 = jnp.float32(1.0 / 256.0)

        def load_xyz(g, resf):
            base = g * 16
            xs = xb[pl.ds(base, 16)] * inv256 * resf
            ys = xb[pl.ds(C + base, 16)] * inv256 * resf
            zs = xb[pl.ds(2 * C + base, 16)] * inv256 * resf
            return xs, ys, zs

        def phase_a(l, s):
            resf = jnp.float32(_RES[l])
            rm1 = jnp.float32(_RES[l] - 1)

            def body(g, carry):
                xs, ys, zs = load_xyz(g, resf)
                cx = xs.astype(jnp.int32)
                cy = ys.astype(jnp.int32)
                cz = zs.astype(jnp.int32)
                if _DENSE[l]:
                    s1 = _RES[l] + 1
                    ax = (cx, cx + 1)
                    by0 = cy * s1
                    by = (by0, by0 + s1)
                    cz0 = cz * (s1 * s1)
                    czc = (cz0, cz0 + s1 * s1)
                    ids = [ax[i] + by[j] + czc[k]
                           for i in (0, 1) for j in (0, 1) for k in (0, 1)]
                else:
                    ux = cx.astype(jnp.uint32)
                    uy = cy.astype(jnp.uint32)
                    uz = cz.astype(jnp.uint32)
                    hx = (ux, ux + jnp.uint32(1))
                    hy0 = uy * jnp.uint32(PRIME1)
                    hy = (hy0, hy0 + jnp.uint32(PRIME1))
                    hz0 = uz * jnp.uint32(PRIME2)
                    hz = (hz0, hz0 + jnp.uint32(PRIME2))
                    mask = jnp.uint32(T - 1)
                    ids = [((hx[i] ^ hy[j] ^ hz[k]) & mask).astype(jnp.int32)
                           for i in (0, 1) for j in (0, 1) for k in (0, 1)]
                for c8 in range(8):
                    idxs[s][pl.ds((g * 8 + c8) * 16, 16)] = ids[c8]
                return carry

            lax.fori_loop(0, G, body, 0)

        def issue(l, s):
            src_ref = sp_refs[l] if l < _N_STAGED else tab_refs[l]
            return pltpu.async_copy(src_ref.at[idxs[s]], lands[s], sems[s])

        def phase_b(l, s):
            resf = jnp.float32(_RES[l])

            def body(g, carry):
                xs, ys, zs = load_xyz(g, resf)
                wx1 = xs - xs.astype(jnp.int32).astype(jnp.float32)
                wy1 = ys - ys.astype(jnp.int32).astype(jnp.float32)
                wz1 = zs - zs.astype(jnp.int32).astype(jnp.float32)
                wx = (jnp.float32(1.0) - wx1, wx1)
                wy = (jnp.float32(1.0) - wy1, wy1)
                wz = (jnp.float32(1.0) - wz1, wz1)
                m = [wx[i] * wy[j] for i in (0, 1) for j in (0, 1)]
                acc0 = jnp.zeros((16,), jnp.float32)
                acc1 = jnp.zeros((16,), jnp.float32)
                rbase = g * 128
                for c8 in range(8):
                    i = (c8 >> 2) & 1
                    j = (c8 >> 1) & 1
                    k = c8 & 1
                    wc = m[i * 2 + j] * wz[k]
                    packed = lands[s][pl.ds(rbase + c8 * 16, 16)]
                    f0 = lax.bitcast_convert_type(lax.shift_left(packed, 16), jnp.float32)
                    f1 = lax.bitcast_convert_type(packed & jnp.int32(-65536), jnp.float32)
                    acc0 = acc0 + f0 * wc
                    acc1 = acc1 + f1 * wc
                peb[2 * l, pl.ds(g * 16, 16)] = acc0
                peb[2 * l + 1, pl.ds(g * 16, 16)] = acc1
                return carry

            lax.fori_loop(0, G, body, 0)

        def chunk_body(ci, carry):
            rowbase = wid * PPW + ci * C
            pltpu.sync_copy(x_refs[0].at[pl.ds(rowbase, C)], xb.at[pl.ds(0, C)])
            pltpu.sync_copy(x_refs[1].at[pl.ds(rowbase, C)], xb.at[pl.ds(C, C)])
            pltpu.sync_copy(x_refs[2].at[pl.ds(rowbase, C)], xb.at[pl.ds(2 * C, C)])
            phase_a(0, 0)
            cp = issue(0, 0)
            for l in range(N_LEVELS):
                s = l % 2
                nxt = None
                if l + 1 < N_LEVELS:
                    phase_a(l + 1, 1 - s)
                    nxt = issue(l + 1, 1 - s)
                cp.wait()
                phase_b(l, s)
                cp = nxt
            pltpu.sync_copy(peb, pe_h.at[:, pl.ds(rowbase, C)])
            return carry

        s_idx = lax.axis_index("s")
        piece = 0
        for l in range(_N_STAGED):
            off = 0
            while off < _STAGE_SZ[l]:
                n = min(8 * C, _STAGE_SZ[l] - off)

                def _stage(l=l, off=off, n=n):
                    pltpu.sync_copy(tab_refs[l].at[pl.ds(off, n)],
                                    land0.at[pl.ds(0, n)])
                    pltpu.sync_copy(land0.at[pl.ds(0, n)],
                                    sp_refs[l].at[pl.ds(off, n)])

                pl.when(s_idx == (piece % NS))(_stage)
                piece += 1
                off += n
        plsc.subcore_barrier()
        lax.fori_loop(0, NCHUNK, chunk_body, 0)

    return enc(x0, x1, x2, *tabs)


def _tc_mlp(peT, W0, W1, W2):
    B = 2048

    def body(pet_ref, w0_ref, w1_ref, w2_ref, pe_ref, z_ref, d_ref):
        p = pet_ref[...].T
        pe_ref[...] = p
        h = jnp.maximum(
            jnp.dot(p, w0_ref[...], preferred_element_type=jnp.float32), 0.0)
        h = jnp.maximum(
            jnp.dot(h, w1_ref[...], preferred_element_type=jnp.float32), 0.0)
        z = jnp.dot(h, w2_ref[...], preferred_element_type=jnp.float32)
        z_ref[...] = z
        z0 = z[:, 0]
        d_ref[...] = jnp.maximum(z0, 0.0) + jnp.log1p(jnp.exp(-jnp.abs(z0)))

    return pl.pallas_call(
        body,
        grid=(N_POINTS // B,),
        in_specs=[
            pl.BlockSpec((2 * N_LEVELS, B), lambda i: (0, i)),
            pl.BlockSpec((2 * N_LEVELS, WIDTH), lambda i: (0, 0)),
            pl.BlockSpec((WIDTH, WIDTH), lambda i: (0, 0)),
            pl.BlockSpec((WIDTH, N_OUT), lambda i: (0, 0)),
        ],
        out_specs=[
            pl.BlockSpec((B, 2 * N_LEVELS), lambda i: (i, 0)),
            pl.BlockSpec((B, N_OUT), lambda i: (i, 0)),
            pl.BlockSpec((B,), lambda i: (i,)),
        ],
        out_shape=[
            jax.ShapeDtypeStruct((N_POINTS, 2 * N_LEVELS), jnp.float32),
            jax.ShapeDtypeStruct((N_POINTS, N_OUT), jnp.float32),
            jax.ShapeDtypeStruct((N_POINTS,), jnp.float32),
        ],
    )(peT, W0, W1, W2)


def kernel(x, tables, W0, W1, W2):
    x0 = x[:, 0]
    x1 = x[:, 1]
    x2 = x[:, 2]
    tabs = [lax.bitcast_convert_type(tables[l].astype(jnp.bfloat16), jnp.int32)
            for l in range(N_LEVELS)]
    peT = _sc_encode(x0, x1, x2, *tabs)
    pe, z, density = _tc_mlp(peT, W0, W1, W2)
    return (density, pe, z)


# speedup vs baseline: 494.4756x; 2.3417x over previous
import functools

import numpy as np
import jax
import jax.numpy as jnp
from jax import lax
from jax.experimental import pallas as pl
from jax.experimental.pallas import tpu as pltpu
from jax.experimental.pallas import tpu_sc as plsc

N_POINTS = 1048576
N_LEVELS = 12
LOG2_T = 19
T = 1 << LOG2_T
BASE_RES = 16
SCALE = 1.3819
WIDTH = 64
N_OUT = 16
PRIME1 = 2654435761
PRIME2 = 805459861

_RES = [int(np.floor(BASE_RES * (SCALE ** l))) for l in range(N_LEVELS)]
_DENSE = [(r + 1) ** 3 <= T for r in _RES]
_ROWS = [min((_RES[l] + 1) ** 3, T) if _DENSE[l] else T for l in range(N_LEVELS)]

NC = 2
NS = 16
NW = NC * NS
PPW = N_POINTS // NW
C = 1024
G = C // 16
SUPER = 16384
NSUP = PPW // SUPER
NCHUNK = SUPER // C
PIECE = 8 * C
_NPIECES = [-(-_ROWS[l] // PIECE) for l in range(N_LEVELS)]
SPSZ = max(n * PIECE for n in _NPIECES)


_N_DENSE = sum(_DENSE)


def _sc_encode(x0, x1, x2, tabcat, resf_a, s1_a, npc_a, npts, nsup):
    mesh = plsc.VectorSubcoreMesh(core_axis_name="c", subcore_axis_name="s")

    @functools.partial(
        pl.kernel,
        out_type=jax.ShapeDtypeStruct((2 * N_LEVELS, npts), jnp.float32),
        mesh=mesh,
        scratch_types=[
            pltpu.VMEM((32,), jnp.float32),
            pltpu.VMEM((32,), jnp.int32),
            pltpu.VMEM((32,), jnp.int32),
            pltpu.VMEM((3 * SUPER,), jnp.float32),
            pltpu.VMEM((8 * C,), jnp.int32),
            pltpu.VMEM((8 * C,), jnp.int32),
            pltpu.VMEM((8 * C,), jnp.int32),
            pltpu.VMEM((8 * C,), jnp.int32),
            pltpu.VMEM((2, C), jnp.float32),
            pltpu.SemaphoreType.DMA,
            pltpu.SemaphoreType.DMA,
        ] + [pltpu.VMEM_SHARED((SPSZ,), jnp.int32)],
    )
    def enc(*refs):
        x_refs = refs[:3]
        tabcat_h, resf_h, s1_h, npc_h, pe_h = refs[3:8]
        (resf_m, s1_m, npc_m, xb, idx0, idx1, land0, land1, peb,
         sem0, sem1, sptab) = refs[8:]
        idxs = (idx0, idx1)
        lands = (land0, land1)
        sems = (sem0, sem1)

        wid = lax.axis_index("s") * NC + lax.axis_index("c")
        s_idx = lax.axis_index("s")
        inv256 = jnp.float32(1.0 / 256.0)
        pbase = wid * (npts // NW)

        def load_xyz(p0, resf):
            xs = xb[pl.ds(p0, 16)] * inv256 * resf
            ys = xb[pl.ds(SUPER + p0, 16)] * inv256 * resf
            zs = xb[pl.ds(2 * SUPER + p0, 16)] * inv256 * resf
            return xs, ys, zs

        def phase_a(l, dense, s, ci):
            resf = resf_m[pl.ds(l, 16)][0]
            s1 = s1_m[pl.ds(l, 16)][0]
            s1sq = s1 * s1

            def body(g):
                xs, ys, zs = load_xyz(ci * C + g * 16, resf)
                cx = xs.astype(jnp.int32)
                cy = ys.astype(jnp.int32)
                cz = zs.astype(jnp.int32)
                if dense:
                    ax = (cx, cx + 1)
                    by0 = cy * s1
                    by = (by0, by0 + s1)
                    cz0 = cz * s1sq
                    czc = (cz0, cz0 + s1sq)
                    ids = [ax[i] + by[j] + czc[k]
                           for i in (0, 1) for j in (0, 1) for k in (0, 1)]
                else:
                    ux = cx.astype(jnp.uint32)
                    uy = cy.astype(jnp.uint32)
                    uz = cz.astype(jnp.uint32)
                    hx = (ux, ux + jnp.uint32(1))
                    hy0 = uy * jnp.uint32(PRIME1)
                    hy = (hy0, hy0 + jnp.uint32(PRIME1))
                    hz0 = uz * jnp.uint32(PRIME2)
                    hz = (hz0, hz0 + jnp.uint32(PRIME2))
                    mask = jnp.uint32(T - 1)
                    ids = [((hx[i] ^ hy[j] ^ hz[k]) & mask).astype(jnp.int32)
                           for i in (0, 1) for j in (0, 1) for k in (0, 1)]
                for c8 in range(8):
                    idxs[s][pl.ds((g * 8 + c8) * 16, 16)] = ids[c8]

            plsc.parallel_loop(0, G, unroll=2)(body)

        def issue(s):
            return pltpu.async_copy(sptab.at[idxs[s]], lands[s], sems[s])

        def wait(s):
            pltpu.make_async_copy(sptab.at[idxs[s]], lands[s], sems[s]).wait()

        def interp(l, s, ci):
            resf = resf_m[pl.ds(l, 16)][0]

            def body(g):
                xs, ys, zs = load_xyz(ci * C + g * 16, resf)
                wx1 = xs - xs.astype(jnp.int32).astype(jnp.float32)
                wy1 = ys - ys.astype(jnp.int32).astype(jnp.float32)
                wz1 = zs - zs.astype(jnp.int32).astype(jnp.float32)
                wx = (jnp.float32(1.0) - wx1, wx1)
                wy = (jnp.float32(1.0) - wy1, wy1)
                wz = (jnp.float32(1.0) - wz1, wz1)
                m = [wx[i] * wy[j] for i in (0, 1) for j in (0, 1)]
                acc0 = jnp.zeros((16,), jnp.float32)
                acc1 = jnp.zeros((16,), jnp.float32)
                rbase = g * 128
                for c8 in range(8):
                    i = (c8 >> 2) & 1
                    j = (c8 >> 1) & 1
                    k = c8 & 1
                    wc = m[i * 2 + j] * wz[k]
                    packed = lands[s][pl.ds(rbase + c8 * 16, 16)]
                    f0 = lax.bitcast_convert_type(lax.shift_left(packed, 16), jnp.float32)
                    f1 = lax.bitcast_convert_type(packed & jnp.int32(-65536), jnp.float32)
                    acc0 = acc0 + f0 * wc
                    acc1 = acc1 + f1 * wc
                peb[0, pl.ds(g * 16, 16)] = acc0
                peb[1, pl.ds(g * 16, 16)] = acc1

            plsc.parallel_loop(0, G, unroll=2)(body)

        NTEC_MAX = -(-max(_NPIECES) // NS)

        def stage(l):
            np_l = npc_m[pl.ds(l, 16)][0]

            def body(i, carry):
                p = i * NS + s_idx

                @pl.when(p < np_l)
                def _copy():
                    off = p * PIECE
                    pltpu.sync_copy(tabcat_h.at[pl.ds(l * T + off, PIECE)], lands[0])
                    pltpu.sync_copy(lands[0], sptab.at[pl.ds(off, PIECE)])

                return carry

            lax.fori_loop(0, NTEC_MAX, body, 0)

        def run_super(l, dense, sc):
            sbase = pbase + sc * SUPER
            pltpu.sync_copy(x_refs[0].at[pl.ds(sbase, SUPER)], xb.at[pl.ds(0, SUPER)])
            pltpu.sync_copy(x_refs[1].at[pl.ds(sbase, SUPER)], xb.at[pl.ds(SUPER, SUPER)])
            pltpu.sync_copy(x_refs[2].at[pl.ds(sbase, SUPER)], xb.at[pl.ds(2 * SUPER, SUPER)])

            def wb(ci):
                pltpu.sync_copy(peb, pe_h.at[pl.ds(2 * l, 2),
                                             pl.ds(sbase + ci * C, C)])

            phase_a(l, dense, 0, jnp.int32(0))
            issue(0)

            def pair_body(i, carry):
                c1 = 2 * i + 1
                phase_a(l, dense, 1, c1)
                issue(1)
                wait(0)
                interp(l, 0, 2 * i)
                wb(2 * i)

                @pl.when(c1 + 1 < NCHUNK)
                def _prep_next():
                    phase_a(l, dense, 0, c1 + 1)
                    issue(0)

                wait(1)
                interp(l, 1, c1)
                wb(c1)
                return carry

            lax.fori_loop(0, NCHUNK // 2, pair_body, 0)

        pltpu.sync_copy(resf_h, resf_m)
        pltpu.sync_copy(s1_h, s1_m)
        pltpu.sync_copy(npc_h, npc_m)

        def level_body(dense):
            def run(l, carry):
                plsc.subcore_barrier()
                stage(l)
                plsc.subcore_barrier()

                def sup_body(sc, c2):
                    run_super(l, dense, sc)
                    return c2

                lax.fori_loop(0, nsup, sup_body, 0)
                return carry

            return run

        lax.fori_loop(0, _N_DENSE, level_body(True), 0)
        lax.fori_loop(_N_DENSE, N_LEVELS, level_body(False), 0)

    return enc(x0, x1, x2, tabcat, resf_a, s1_a, npc_a)


def _tc_mlp(peT, W0T, W1T, W2T, npts):
    B = 2048

    def body(pet_ref, w0_ref, w1_ref, w2_ref, z_ref, d_ref):
        pT = pet_ref[...]
        hT = jnp.maximum(
            jnp.dot(w0_ref[...], pT, preferred_element_type=jnp.float32), 0.0)
        hT = jnp.maximum(
            jnp.dot(w1_ref[...], hT, preferred_element_type=jnp.float32), 0.0)
        zT = jnp.dot(w2_ref[...], hT, preferred_element_type=jnp.float32)
        z_ref[...] = zT
        z0 = zT[0, :]
        d_ref[...] = jnp.maximum(z0, 0.0) + jnp.log1p(jnp.exp(-jnp.abs(z0)))

    return pl.pallas_call(
        body,
        grid=(npts // B,),
        in_specs=[
            pl.BlockSpec((2 * N_LEVELS, B), lambda i: (0, i)),
            pl.BlockSpec((WIDTH, 2 * N_LEVELS), lambda i: (0, 0)),
            pl.BlockSpec((WIDTH, WIDTH), lambda i: (0, 0)),
            pl.BlockSpec((N_OUT, WIDTH), lambda i: (0, 0)),
        ],
        out_specs=[
            pl.BlockSpec((N_OUT, B), lambda i: (0, i)),
            pl.BlockSpec((B,), lambda i: (i,)),
        ],
        out_shape=[
            jax.ShapeDtypeStruct((N_OUT, npts), jnp.float32),
            jax.ShapeDtypeStruct((npts,), jnp.float32),
        ],
    )(peT, W0T, W1T, W2T)


def kernel(x, tables, W0, W1, W2):
    x0 = x[:, 0]
    x1 = x[:, 1]
    x2 = x[:, 2]
    tabcat = lax.bitcast_convert_type(tables.astype(jnp.bfloat16),
                                      jnp.int32).reshape(-1)
    resf_a = jnp.asarray(np.pad(np.array(_RES, np.float32), (0, 20)))
    s1_a = jnp.asarray(np.pad(np.array([r + 1 for r in _RES], np.int32), (0, 20)))
    npc_a = jnp.asarray(np.pad(np.array(_NPIECES, np.int32), (0, 20)))
    h = N_POINTS // 2
    outs = []
    for lo in (0, h):
        peT = _sc_encode(x0[lo:lo + h], x1[lo:lo + h], x2[lo:lo + h],
                         tabcat, resf_a, s1_a, npc_a, h, NSUP // 2)
        zT, density = _tc_mlp(peT, W0.T, W1.T, W2.T, h)
        outs.append((density, peT.T, zT.T))
    density = jnp.concatenate([o[0] for o in outs])
    pe = jnp.concatenate([o[1] for o in outs])
    z = jnp.concatenate([o[2] for o in outs])
    return (density, pe, z)
